# Initial kernel scaffold; baseline (speedup 1.0000x reference)
import jax
import jax.numpy as jnp
from jax.experimental import pallas as pl


def kernel(x, edge_index, W1, att_src1, att_dst1, b1, W2, att_src2, att_dst2, b2):
    raise NotImplementedError("write your pallas kernel here")


# speedup vs baseline: 8.7050x; 8.7050x over previous
import jax
import jax.numpy as jnp
from jax import lax
from jax.experimental import pallas as pl
from jax.experimental.pallas import tpu as pltpu
from jax.experimental.pallas import tpu_sc as plsc

N = 10000
D = 256
HALF = 128
DROWS = 128
E = 160000
K = 64
NCH = E // K
R = 1024
NRB = (N + R - 1) // R
NC = 2
NS = 16
NPAD = 10240
ROWS_PER_TILE = NPAD // NS
WB = 64
EPS = 1e-16


def _write_att(a_ref, h, as_ref, ad_ref):
    i = pl.program_id(0)
    a_s = jnp.sum(h * as_ref[0][None, :], axis=1)
    a_d = jnp.sum(h * ad_ref[0][None, :], axis=1)
    a_ref[:, pl.ds(i * R, R)] = jnp.concatenate(
        [a_s[None], a_d[None], jnp.zeros((6, R), jnp.float32)], axis=0)


def _dense_in_body(x_ref, w_ref, as_ref, ad_ref, h0_ref, h1_ref, a_ref):
    h = jnp.dot(x_ref[...], w_ref[...], preferred_element_type=jnp.float32)
    h0_ref[...] = h[:, :HALF]
    h1_ref[...] = h[:, HALF:]
    _write_att(a_ref, h, as_ref, ad_ref)


def _dense_in(x, W, att_s, att_d):
    return pl.pallas_call(
        _dense_in_body,
        grid=(NRB,),
        in_specs=[
            pl.BlockSpec((R, D), lambda i: (i, 0)),
            pl.BlockSpec((D, D), lambda i: (0, 0)),
            pl.BlockSpec((1, D), lambda i: (0, 0)),
            pl.BlockSpec((1, D), lambda i: (0, 0)),
        ],
        out_specs=[
            pl.BlockSpec((R, HALF), lambda i: (i, 0)),
            pl.BlockSpec((R, HALF), lambda i: (i, 0)),
            pl.BlockSpec((8, NPAD), lambda i: (0, 0)),
        ],
        out_shape=[
            jax.ShapeDtypeStruct((N, HALF), jnp.float32),
            jax.ShapeDtypeStruct((N, HALF), jnp.float32),
            jax.ShapeDtypeStruct((8, NPAD), jnp.float32),
        ],
    )(x, W, att_s.reshape(1, D), att_d.reshape(1, D))


def _normalize(m_ref, d_ref, b_ref):
    m = jnp.concatenate([m_ref[0], m_ref[1]], axis=1)
    m3 = jnp.reshape(m, (R // HALF, HALF, D))
    d3 = d_ref[0][:, :, None] + EPS
    return jnp.reshape(m3 / d3, (R, D)) + b_ref[0][None, :]


def _dense_mid_body(m_ref, d_ref, b_ref, w_ref, as_ref, ad_ref,
                    h0_ref, h1_ref, a_ref):
    hin = jnp.maximum(_normalize(m_ref, d_ref, b_ref), 0.0)
    h = jnp.dot(hin, w_ref[...], preferred_element_type=jnp.float32)
    h0_ref[...] = h[:, :HALF]
    h1_ref[...] = h[:, HALF:]
    _write_att(a_ref, h, as_ref, ad_ref)


def _dense_mid(msg, den, b, W, att_s, att_d):
    return pl.pallas_call(
        _dense_mid_body,
        grid=(NRB,),
        in_specs=[
            pl.BlockSpec((NC, R, HALF), lambda i: (0, i, 0)),
            pl.BlockSpec((1, R // HALF, HALF), lambda i: (0, i, 0)),
            pl.BlockSpec((1, D), lambda i: (0, 0)),
            pl.BlockSpec((D, D), lambda i: (0, 0)),
            pl.BlockSpec((1, D), lambda i: (0, 0)),
            pl.BlockSpec((1, D), lambda i: (0, 0)),
        ],
        out_specs=[
            pl.BlockSpec((R, HALF), lambda i: (i, 0)),
            pl.BlockSpec((R, HALF), lambda i: (i, 0)),
            pl.BlockSpec((8, NPAD), lambda i: (0, 0)),
        ],
        out_shape=[
            jax.ShapeDtypeStruct((N, HALF), jnp.float32),
            jax.ShapeDtypeStruct((N, HALF), jnp.float32),
            jax.ShapeDtypeStruct((8, NPAD), jnp.float32),
        ],
    )(msg, den, b.reshape(1, D), W, att_s.reshape(1, D), att_d.reshape(1, D))


def _dense_out_body(m_ref, d_ref, b_ref, o_ref):
    o_ref[...] = _normalize(m_ref, d_ref, b_ref)


def _dense_out(msg, den, b):
    return pl.pallas_call(
        _dense_out_body,
        grid=(NRB,),
        in_specs=[
            pl.BlockSpec((NC, R, HALF), lambda i: (0, i, 0)),
            pl.BlockSpec((1, R // HALF, HALF), lambda i: (0, i, 0)),
            pl.BlockSpec((1, D), lambda i: (0, 0)),
        ],
        out_specs=pl.BlockSpec((R, D), lambda i: (i, 0)),
        out_shape=jax.ShapeDtypeStruct((N, D), jnp.float32),
    )(msg, den, b.reshape(1, D))


def _sc_body(h0, h1, a, ei, msg, den,
             asv, adv, srcv, dstv, drow, gbuf, dstg, acc, dacc, sem):
    c = lax.axis_index("c")
    s = lax.axis_index("s")

    pltpu.sync_copy(a.at[0], asv)
    pltpu.sync_copy(a.at[1], adv)

    def zrow(i, carry):
        z = jnp.zeros((16,), jnp.float32)
        for t in range(HALF // 16):
            dstg[i, pl.ds(t * 16, 16)] = z
        return carry

    lax.fori_loop(0, K, zrow, 0)
    base = s * ROWS_PER_TILE
    for q in range(ROWS_PER_TILE // WB):
        pltpu.sync_copy(dstg.at[pl.ds(0, WB)],
                        acc.at[pl.ds(base + q * WB, WB)])
    pltpu.sync_copy(dstg.at[pl.ds(0, DROWS // NS)],
                    dacc.at[pl.ds(s * (DROWS // NS), DROWS // NS)])
    plsc.subcore_barrier()

    def chunk_body(j, carry):
        cid = j * NS + s

        @pl.when(cid < NCH)
        def _():
            e0 = cid * K
            pltpu.sync_copy(ei.at[0, pl.ds(e0, K)], srcv)
            pltpu.sync_copy(ei.at[1, pl.ds(e0, K)], dstv)

            @pl.when(c == 0)
            def _():
                pltpu.async_copy(h0.at[srcv], gbuf, sem).wait()

            @pl.when(c == 1)
            def _():
                pltpu.async_copy(h1.at[srcv], gbuf, sem).wait()

            lanes = lax.iota(jnp.int32, 16)
            z = jnp.zeros((16,), jnp.float32)

            def scale(g, carry2):
                si = srcv[pl.ds(g * 16, 16)]
                di = dstv[pl.ds(g * 16, 16)]
                drow[pl.ds(g * 16, 16)] = di >> 7
                e = plsc.load_gather(asv, [si]) + plsc.load_gather(adv, [di])
                e = jnp.where(e >= 0.0, e, 0.2 * e)
                svec = jnp.exp(e)
                for l in range(16):
                    k2 = g * 16 + l
                    sk = svec[l]
                    for t in range(HALF // 16):
                        gbuf[k2, pl.ds(t * 16, 16)] = (
                            gbuf[k2, pl.ds(t * 16, 16)] * sk)
                    dl = di[l]
                    for t in range(HALF // 16):
                        dstg[k2, pl.ds(t * 16, 16)] = z
                    grp = (dl >> 4) & 7
                    sm = jnp.where(lanes == (dl & 15), sk, 0.0)
                    dstg[k2, pl.ds(grp * 16, 16)] = sm
                return carry2

            lax.fori_loop(0, K // 16, scale, 0)
            pltpu.sync_copy(gbuf, acc.at[dstv], add=True)
            pltpu.sync_copy(dstg, dacc.at[drow], add=True)

        return carry

    lax.fori_loop(0, (NCH + NS - 1) // NS, chunk_body, 0)
    plsc.subcore_barrier()

    for q in range(ROWS_PER_TILE // WB):
        r0 = base + q * WB
        pltpu.sync_copy(acc.at[pl.ds(r0, WB)], gbuf.at[pl.ds(0, WB)])
        pltpu.sync_copy(gbuf.at[pl.ds(0, WB)], msg.at[c, pl.ds(r0, WB)])
    d0 = s * (DROWS // NS)
    pltpu.sync_copy(dacc.at[pl.ds(d0, DROWS // NS)],
                    dstg.at[pl.ds(0, DROWS // NS)])
    pltpu.sync_copy(dstg.at[pl.ds(0, DROWS // NS)],
                    den.at[c, pl.ds(d0, DROWS // NS)])


def _sc_edge(h0, h1, a, ei):
    mesh = plsc.VectorSubcoreMesh(
        core_axis_name="c", subcore_axis_name="s",
        num_cores=NC, num_subcores=NS)
    kern = pl.kernel(
        _sc_body,
        out_type=[
            jax.ShapeDtypeStruct((NC, NPAD, HALF), jnp.float32),
            jax.ShapeDtypeStruct((NC, DROWS, HALF), jnp.float32),
        ],
        mesh=mesh,
        compiler_params=pltpu.CompilerParams(needs_layout_passes=False),
        scratch_types=[
            pltpu.VMEM((NPAD,), jnp.float32),
            pltpu.VMEM((NPAD,), jnp.float32),
            pltpu.VMEM((K,), jnp.int32),
            pltpu.VMEM((K,), jnp.int32),
            pltpu.VMEM((K,), jnp.int32),
            pltpu.VMEM((K, HALF), jnp.float32),
            pltpu.VMEM((K, HALF), jnp.float32),
            pltpu.VMEM_SHARED((NPAD, HALF), jnp.float32),
            pltpu.VMEM_SHARED((DROWS, HALF), jnp.float32),
            pltpu.SemaphoreType.DMA,
        ],
    )
    return kern(h0, h1, a, ei)


def kernel(x, edge_index, W1, att_src1, att_dst1, b1,
           W2, att_src2, att_dst2, b2):
    ei = edge_index.astype(jnp.int32)
    h0, h1, a1 = _dense_in(x, W1, att_src1, att_dst1)
    msg1, den1 = _sc_edge(h0, h1, a1, ei)
    h0b, h1b, a2 = _dense_mid(msg1, den1, b1, W2, att_src2, att_dst2)
    msg2, den2 = _sc_edge(h0b, h1b, a2, ei)
    return _dense_out(msg2, den2, b2)


# speedup vs baseline: 9.2455x; 1.0621x over previous
import jax
import jax.numpy as jnp
from jax import lax
from jax.experimental import pallas as pl
from jax.experimental.pallas import tpu as pltpu
from jax.experimental.pallas import tpu_sc as plsc

N = 10000
D = 256
HALF = 128
DROWS = 128
E = 160000
K = 64
NCH = E // K
R = 1024
NRB = (N + R - 1) // R
NC = 2
NS = 16
NPAD = 10240
ROWS_PER_TILE = NPAD // NS
WB = 64
EPS = 1e-16


def _write_att(a_ref, h, as_ref, ad_ref):
    i = pl.program_id(0)
    a_s = jnp.sum(h * as_ref[0][None, :], axis=1)
    a_d = jnp.sum(h * ad_ref[0][None, :], axis=1)
    a_ref[:, pl.ds(i * R, R)] = jnp.concatenate(
        [a_s[None], a_d[None], jnp.zeros((6, R), jnp.float32)], axis=0)


def _dense_in_body(x_ref, w_ref, as_ref, ad_ref, h0_ref, h1_ref, a_ref):
    h = jnp.dot(x_ref[...], w_ref[...], preferred_element_type=jnp.float32)
    h0_ref[...] = h[:, :HALF]
    h1_ref[...] = h[:, HALF:]
    _write_att(a_ref, h, as_ref, ad_ref)


def _dense_in(x, W, att_s, att_d):
    return pl.pallas_call(
        _dense_in_body,
        grid=(NRB,),
        in_specs=[
            pl.BlockSpec((R, D), lambda i: (i, 0)),
            pl.BlockSpec((D, D), lambda i: (0, 0)),
            pl.BlockSpec((1, D), lambda i: (0, 0)),
            pl.BlockSpec((1, D), lambda i: (0, 0)),
        ],
        out_specs=[
            pl.BlockSpec((R, HALF), lambda i: (i, 0)),
            pl.BlockSpec((R, HALF), lambda i: (i, 0)),
            pl.BlockSpec((8, NPAD), lambda i: (0, 0)),
        ],
        out_shape=[
            jax.ShapeDtypeStruct((N, HALF), jnp.float32),
            jax.ShapeDtypeStruct((N, HALF), jnp.float32),
            jax.ShapeDtypeStruct((8, NPAD), jnp.float32),
        ],
    )(x, W, att_s.reshape(1, D), att_d.reshape(1, D))


def _normalize(m_ref, d_ref, b_ref):
    m = jnp.concatenate([m_ref[0], m_ref[1]], axis=1)
    m3 = jnp.reshape(m, (R // HALF, HALF, D))
    d3 = d_ref[0][:, :, None] + EPS
    return jnp.reshape(m3 / d3, (R, D)) + b_ref[0][None, :]


def _dense_mid_body(m_ref, d_ref, b_ref, w_ref, as_ref, ad_ref,
                    h0_ref, h1_ref, a_ref):
    hin = jnp.maximum(_normalize(m_ref, d_ref, b_ref), 0.0)
    h = jnp.dot(hin, w_ref[...], preferred_element_type=jnp.float32)
    h0_ref[...] = h[:, :HALF]
    h1_ref[...] = h[:, HALF:]
    _write_att(a_ref, h, as_ref, ad_ref)


def _dense_mid(msg, den, b, W, att_s, att_d):
    return pl.pallas_call(
        _dense_mid_body,
        grid=(NRB,),
        in_specs=[
            pl.BlockSpec((NC, R, HALF), lambda i: (0, i, 0)),
            pl.BlockSpec((1, R // HALF, HALF), lambda i: (0, i, 0)),
            pl.BlockSpec((1, D), lambda i: (0, 0)),
            pl.BlockSpec((D, D), lambda i: (0, 0)),
            pl.BlockSpec((1, D), lambda i: (0, 0)),
            pl.BlockSpec((1, D), lambda i: (0, 0)),
        ],
        out_specs=[
            pl.BlockSpec((R, HALF), lambda i: (i, 0)),
            pl.BlockSpec((R, HALF), lambda i: (i, 0)),
            pl.BlockSpec((8, NPAD), lambda i: (0, 0)),
        ],
        out_shape=[
            jax.ShapeDtypeStruct((N, HALF), jnp.float32),
            jax.ShapeDtypeStruct((N, HALF), jnp.float32),
            jax.ShapeDtypeStruct((8, NPAD), jnp.float32),
        ],
    )(msg, den, b.reshape(1, D), W, att_s.reshape(1, D), att_d.reshape(1, D))


def _dense_out_body(m_ref, d_ref, b_ref, o_ref):
    o_ref[...] = _normalize(m_ref, d_ref, b_ref)


def _dense_out(msg, den, b):
    return pl.pallas_call(
        _dense_out_body,
        grid=(NRB,),
        in_specs=[
            pl.BlockSpec((NC, R, HALF), lambda i: (0, i, 0)),
            pl.BlockSpec((1, R // HALF, HALF), lambda i: (0, i, 0)),
            pl.BlockSpec((1, D), lambda i: (0, 0)),
        ],
        out_specs=pl.BlockSpec((R, D), lambda i: (i, 0)),
        out_shape=jax.ShapeDtypeStruct((N, D), jnp.float32),
    )(msg, den, b.reshape(1, D))


EPAD = NS * 160 * K
CPB = 16
IB = CPB * K


def _sc_body(h0, h1, a, ei, msg, den,
             asv, adv, sbuf, dbuf, srcv, dstv, drow, sv, pgrp,
             gbuf, dstg, acc, dacc, sem):
    c = lax.axis_index("c")
    s = lax.axis_index("s")

    pltpu.sync_copy(a.at[0], asv)
    pltpu.sync_copy(a.at[1], adv)

    def zrow(i, carry):
        z = jnp.zeros((16,), jnp.float32)
        for t in range(HALF // 16):
            dstg[i, pl.ds(t * 16, 16)] = z
        return carry

    lax.fori_loop(0, K, zrow, 0)
    for t in range(K // 16):
        pgrp[pl.ds(t * 16, 16)] = jnp.zeros((16,), jnp.int32)
    base = s * ROWS_PER_TILE
    for q in range(ROWS_PER_TILE // WB):
        pltpu.sync_copy(dstg.at[pl.ds(0, WB)],
                        acc.at[pl.ds(base + q * WB, WB)])
    pltpu.sync_copy(dstg.at[pl.ds(0, DROWS // NS)],
                    dacc.at[pl.ds(s * (DROWS // NS), DROWS // NS)])
    plsc.subcore_barrier()

    ebase = s * (160 * K)

    def load_batch(b):
        e0 = ebase + b * IB
        pltpu.sync_copy(ei.at[0, pl.ds(e0, IB)], sbuf)
        pltpu.sync_copy(ei.at[1, pl.ds(e0, IB)], dbuf)

    def stage_src(jj):
        off = (jj % CPB) * K
        for t in range(K // 16):
            srcv[pl.ds(t * 16, 16)] = sbuf[pl.ds(off + t * 16, 16)]

    load_batch(0)
    stage_src(0)

    @pl.when(c == 0)
    def _():
        pltpu.async_copy(h0.at[srcv], gbuf, sem, add=False)

    @pl.when(c == 1)
    def _():
        pltpu.async_copy(h1.at[srcv], gbuf, sem, add=False)

    lanes = lax.iota(jnp.int32, 16)
    z16 = jnp.zeros((16,), jnp.float32)

    def chunk_body(j, carry):
        off = (j % CPB) * K

        def prep(g, carry2):
            si = srcv[pl.ds(g * 16, 16)]
            di = dbuf[pl.ds(off + g * 16, 16)]
            dstv[pl.ds(g * 16, 16)] = di
            drow[pl.ds(g * 16, 16)] = di >> 7
            e = plsc.load_gather(asv, [si]) + plsc.load_gather(adv, [di])
            e = jnp.where(e >= 0.0, e, 0.2 * e)
            svec = jnp.exp(e)
            sv[pl.ds(g * 16, 16)] = svec
            og = pgrp[pl.ds(g * 16, 16)]
            ng = (di >> 4) & 7
            pgrp[pl.ds(g * 16, 16)] = ng
            for l in range(16):
                k2 = g * 16 + l
                dstg[k2, pl.ds(og[l] * 16, 16)] = z16
                sm = jnp.where(lanes == (di[l] & 15), svec[l], 0.0)
                dstg[k2, pl.ds(ng[l] * 16, 16)] = sm
            return carry2

        lax.fori_loop(0, K // 16, prep, 0)

        pltpu.make_async_copy(h0.at[srcv], gbuf, sem).wait()

        def scale(g, carry2):
            svec = sv[pl.ds(g * 16, 16)]
            for l in range(16):
                k2 = g * 16 + l
                sk = svec[l]
                for t in range(HALF // 16):
                    gbuf[k2, pl.ds(t * 16, 16)] = (
                        gbuf[k2, pl.ds(t * 16, 16)] * sk)
            return carry2

        lax.fori_loop(0, K // 16, scale, 0)

        pltpu.sync_copy(gbuf, acc.at[dstv], add=True)

        @pl.when(j < 160 * 1 - 1)
        def _():
            @pl.when((j + 1) % CPB == 0)
            def _():
                load_batch((j + 1) // CPB)

            stage_src(j + 1)

            @pl.when(c == 0)
            def _():
                pltpu.async_copy(h0.at[srcv], gbuf, sem, add=False)

            @pl.when(c == 1)
            def _():
                pltpu.async_copy(h1.at[srcv], gbuf, sem, add=False)

        pltpu.sync_copy(dstg, dacc.at[drow], add=True)
        return carry

    lax.fori_loop(0, 160, chunk_body, 0)
    plsc.subcore_barrier()

    for q in range(ROWS_PER_TILE // WB):
        r0 = base + q * WB
        pltpu.sync_copy(acc.at[pl.ds(r0, WB)], gbuf.at[pl.ds(0, WB)])
        pltpu.sync_copy(gbuf.at[pl.ds(0, WB)], msg.at[c, pl.ds(r0, WB)])
    d0 = s * (DROWS // NS)
    pltpu.sync_copy(dacc.at[pl.ds(d0, DROWS // NS)],
                    dstg.at[pl.ds(0, DROWS // NS)])
    pltpu.sync_copy(dstg.at[pl.ds(0, DROWS // NS)],
                    den.at[c, pl.ds(d0, DROWS // NS)])


def _sc_edge(h0, h1, a, ei):
    mesh = plsc.VectorSubcoreMesh(
        core_axis_name="c", subcore_axis_name="s",
        num_cores=NC, num_subcores=NS)
    kern = pl.kernel(
        _sc_body,
        out_type=[
            jax.ShapeDtypeStruct((NC, NPAD, HALF), jnp.float32),
            jax.ShapeDtypeStruct((NC, DROWS, HALF), jnp.float32),
        ],
        mesh=mesh,
        compiler_params=pltpu.CompilerParams(needs_layout_passes=False),
        scratch_types=[
            pltpu.VMEM((NPAD,), jnp.float32),
            pltpu.VMEM((NPAD,), jnp.float32),
            pltpu.VMEM((IB,), jnp.int32),
            pltpu.VMEM((IB,), jnp.int32),
            pltpu.VMEM((K,), jnp.int32),
            pltpu.VMEM((K,), jnp.int32),
            pltpu.VMEM((K,), jnp.int32),
            pltpu.VMEM((K,), jnp.float32),
            pltpu.VMEM((K,), jnp.int32),
            pltpu.VMEM((K, HALF), jnp.float32),
            pltpu.VMEM((K, HALF), jnp.float32),
            pltpu.VMEM_SHARED((NPAD, HALF), jnp.float32),
            pltpu.VMEM_SHARED((DROWS, HALF), jnp.float32),
            pltpu.SemaphoreType.DMA,
        ],
    )
    return kern(h0, h1, a, ei)


def kernel(x, edge_index, W1, att_src1, att_dst1, b1,
           W2, att_src2, att_dst2, b2):
    ei = edge_index.astype(jnp.int32)
    pad = jnp.concatenate(
        [jnp.zeros((1, EPAD - E), jnp.int32),
         jnp.full((1, EPAD - E), NPAD - 1, jnp.int32)], axis=0)
    ei = jnp.concatenate([ei, pad], axis=1)
    h0, h1, a1 = _dense_in(x, W1, att_src1, att_dst1)
    msg1, den1 = _sc_edge(h0, h1, a1, ei)
    h0b, h1b, a2 = _dense_mid(msg1, den1, b1, W2, att_src2, att_dst2)
    msg2, den2 = _sc_edge(h0b, h1b, a2, ei)
    return _dense_out(msg2, den2, b2)


# speedup vs baseline: 9.4117x; 1.0180x over previous
import jax
import jax.numpy as jnp
from jax import lax
from jax.experimental import pallas as pl
from jax.experimental.pallas import tpu as pltpu
from jax.experimental.pallas import tpu_sc as plsc

N = 10000
D = 256
HALF = 128
DROWS = 128
E = 160000
K = 64
NCH = E // K
R = 1024
NRB = (N + R - 1) // R
NC = 2
NS = 16
NPAD = 10240
ROWS_PER_TILE = NPAD // NS
WB = 64
EPS = 1e-16


def _write_att(a_ref, h, as_ref, ad_ref):
    i = pl.program_id(0)
    a_s = jnp.sum(h * as_ref[0][None, :], axis=1)
    a_d = jnp.sum(h * ad_ref[0][None, :], axis=1)
    a_ref[:, pl.ds(i * R, R)] = jnp.concatenate(
        [a_s[None], a_d[None], jnp.zeros((6, R), jnp.float32)], axis=0)


def _dense_in_body(x_ref, w_ref, as_ref, ad_ref, h0_ref, h1_ref, a_ref):
    h = jnp.dot(x_ref[...], w_ref[...], preferred_element_type=jnp.float32)
    h0_ref[...] = h[:, :HALF]
    h1_ref[...] = h[:, HALF:]
    _write_att(a_ref, h, as_ref, ad_ref)


def _dense_in(x, W, att_s, att_d):
    return pl.pallas_call(
        _dense_in_body,
        grid=(NRB,),
        in_specs=[
            pl.BlockSpec((R, D), lambda i: (i, 0)),
            pl.BlockSpec((D, D), lambda i: (0, 0)),
            pl.BlockSpec((1, D), lambda i: (0, 0)),
            pl.BlockSpec((1, D), lambda i: (0, 0)),
        ],
        out_specs=[
            pl.BlockSpec((R, HALF), lambda i: (i, 0)),
            pl.BlockSpec((R, HALF), lambda i: (i, 0)),
            pl.BlockSpec((8, NPAD), lambda i: (0, 0)),
        ],
        out_shape=[
            jax.ShapeDtypeStruct((N, HALF), jnp.float32),
            jax.ShapeDtypeStruct((N, HALF), jnp.float32),
            jax.ShapeDtypeStruct((8, NPAD), jnp.float32),
        ],
    )(x, W, att_s.reshape(1, D), att_d.reshape(1, D))


def _normalize(m_ref, d_ref, b_ref):
    m = jnp.concatenate([m_ref[0], m_ref[1]], axis=1)
    m3 = jnp.reshape(m, (R // HALF, HALF, D))
    d3 = d_ref[0][:, :, None] + EPS
    return jnp.reshape(m3 / d3, (R, D)) + b_ref[0][None, :]


def _dense_mid_body(m_ref, d_ref, b_ref, w_ref, as_ref, ad_ref,
                    h0_ref, h1_ref, a_ref):
    hin = jnp.maximum(_normalize(m_ref, d_ref, b_ref), 0.0)
    h = jnp.dot(hin, w_ref[...], preferred_element_type=jnp.float32)
    h0_ref[...] = h[:, :HALF]
    h1_ref[...] = h[:, HALF:]
    _write_att(a_ref, h, as_ref, ad_ref)


def _dense_mid(msg, den, b, W, att_s, att_d):
    return pl.pallas_call(
        _dense_mid_body,
        grid=(NRB,),
        in_specs=[
            pl.BlockSpec((NC, R, HALF), lambda i: (0, i, 0)),
            pl.BlockSpec((1, R // HALF, HALF), lambda i: (0, i, 0)),
            pl.BlockSpec((1, D), lambda i: (0, 0)),
            pl.BlockSpec((D, D), lambda i: (0, 0)),
            pl.BlockSpec((1, D), lambda i: (0, 0)),
            pl.BlockSpec((1, D), lambda i: (0, 0)),
        ],
        out_specs=[
            pl.BlockSpec((R, HALF), lambda i: (i, 0)),
            pl.BlockSpec((R, HALF), lambda i: (i, 0)),
            pl.BlockSpec((8, NPAD), lambda i: (0, 0)),
        ],
        out_shape=[
            jax.ShapeDtypeStruct((N, HALF), jnp.float32),
            jax.ShapeDtypeStruct((N, HALF), jnp.float32),
            jax.ShapeDtypeStruct((8, NPAD), jnp.float32),
        ],
    )(msg, den, b.reshape(1, D), W, att_s.reshape(1, D), att_d.reshape(1, D))


def _dense_out_body(m_ref, d_ref, b_ref, o_ref):
    o_ref[...] = _normalize(m_ref, d_ref, b_ref)


def _dense_out(msg, den, b):
    return pl.pallas_call(
        _dense_out_body,
        grid=(NRB,),
        in_specs=[
            pl.BlockSpec((NC, R, HALF), lambda i: (0, i, 0)),
            pl.BlockSpec((1, R // HALF, HALF), lambda i: (0, i, 0)),
            pl.BlockSpec((1, D), lambda i: (0, 0)),
        ],
        out_specs=pl.BlockSpec((R, D), lambda i: (i, 0)),
        out_shape=jax.ShapeDtypeStruct((N, D), jnp.float32),
    )(msg, den, b.reshape(1, D))


EPAD = NS * 160 * K
CPB = 16
IB = CPB * K


def _sc_body(h0, h1, a, ei, msg, den,
             asv, adv, sbuf, dbuf, srcv, dstv2, drow2, sv, pgrp,
             gbuf, dstg, acc, dacc, semg, sema, semd):
    c = lax.axis_index("c")
    s = lax.axis_index("s")

    pltpu.sync_copy(a.at[0], asv)
    pltpu.sync_copy(a.at[1], adv)

    def zrow(i, carry):
        z = jnp.zeros((16,), jnp.float32)
        for t in range(HALF // 16):
            dstg[i, pl.ds(t * 16, 16)] = z
        return carry

    lax.fori_loop(0, K, zrow, 0)
    for t in range(K // 16):
        pgrp[pl.ds(t * 16, 16)] = jnp.zeros((16,), jnp.int32)
    base = s * ROWS_PER_TILE
    for q in range(ROWS_PER_TILE // WB):
        pltpu.sync_copy(dstg.at[pl.ds(0, WB)],
                        acc.at[pl.ds(base + q * WB, WB)])
    pltpu.sync_copy(dstg.at[pl.ds(0, DROWS // NS)],
                    dacc.at[pl.ds(s * (DROWS // NS), DROWS // NS)])
    plsc.subcore_barrier()

    ebase = s * (160 * K)

    def load_batch(b):
        e0 = ebase + b * IB
        pltpu.sync_copy(ei.at[0, pl.ds(e0, IB)], sbuf)
        pltpu.sync_copy(ei.at[1, pl.ds(e0, IB)], dbuf)

    def stage_src(jj):
        off = (jj % CPB) * K
        for t in range(K // 16):
            srcv[pl.ds(t * 16, 16)] = sbuf[pl.ds(off + t * 16, 16)]

    load_batch(0)
    stage_src(0)
    for t in range(K // 16):
        drow2[1, pl.ds(t * 16, 16)] = jnp.zeros((16,), jnp.int32)

    @pl.when(c == 0)
    def _():
        pltpu.async_copy(h0.at[srcv], gbuf, semg, add=False)

    @pl.when(c == 1)
    def _():
        pltpu.async_copy(h1.at[srcv], gbuf, semg, add=False)

    pltpu.async_copy(dstg, dacc.at[drow2.at[1]], semd, add=True)

    lanes = lax.iota(jnp.int32, 16)
    z16 = jnp.zeros((16,), jnp.float32)

    def chunk_body(j, carry):
        off = (j % CPB) * K
        p = j & 1

        pltpu.make_async_copy(dstg, dacc.at[drow2.at[1 - p]], semd).wait()

        def prep(g, carry2):
            si = srcv[pl.ds(g * 16, 16)]
            di = dbuf[pl.ds(off + g * 16, 16)]
            dstv2[p, pl.ds(g * 16, 16)] = di
            drow2[p, pl.ds(g * 16, 16)] = di >> 7
            e = plsc.load_gather(asv, [si]) + plsc.load_gather(adv, [di])
            e = jnp.where(e >= 0.0, e, 0.2 * e)
            svec = jnp.exp(e)
            sv[pl.ds(g * 16, 16)] = svec
            og = pgrp[pl.ds(g * 16, 16)]
            ng = (di >> 4) & 7
            pgrp[pl.ds(g * 16, 16)] = ng
            for l in range(16):
                k2 = g * 16 + l
                dstg[k2, pl.ds(og[l] * 16, 16)] = z16
                sm = jnp.where(lanes == (di[l] & 15), svec[l], 0.0)
                dstg[k2, pl.ds(ng[l] * 16, 16)] = sm
            return carry2

        lax.fori_loop(0, K // 16, prep, 0)

        pltpu.async_copy(dstg, dacc.at[drow2.at[p]], semd, add=True)

        pltpu.make_async_copy(h0.at[srcv], gbuf, semg).wait()

        def scale(g, carry2):
            svec = sv[pl.ds(g * 16, 16)]
            for l in range(16):
                k2 = g * 16 + l
                sk = svec[l]
                for t in range(HALF // 16):
                    gbuf[k2, pl.ds(t * 16, 16)] = (
                        gbuf[k2, pl.ds(t * 16, 16)] * sk)
            return carry2

        lax.fori_loop(0, K // 16, scale, 0)

        pltpu.async_copy(gbuf, acc.at[dstv2.at[p]], sema, add=True)

        @pl.when(j < 159)
        def _():
            @pl.when((j + 1) % CPB == 0)
            def _():
                load_batch((j + 1) // CPB)

            stage_src(j + 1)

        pltpu.make_async_copy(gbuf, acc.at[dstv2.at[p]], sema).wait()

        @pl.when(j < 159)
        def _():
            @pl.when(c == 0)
            def _():
                pltpu.async_copy(h0.at[srcv], gbuf, semg, add=False)

            @pl.when(c == 1)
            def _():
                pltpu.async_copy(h1.at[srcv], gbuf, semg, add=False)

        return carry

    lax.fori_loop(0, 160, chunk_body, 0)
    pltpu.make_async_copy(dstg, dacc.at[drow2.at[1]], semd).wait()
    plsc.subcore_barrier()

    for q in range(ROWS_PER_TILE // WB):
        r0 = base + q * WB
        pltpu.sync_copy(acc.at[pl.ds(r0, WB)], gbuf.at[pl.ds(0, WB)])
        pltpu.sync_copy(gbuf.at[pl.ds(0, WB)], msg.at[c, pl.ds(r0, WB)])
    d0 = s * (DROWS // NS)
    pltpu.sync_copy(dacc.at[pl.ds(d0, DROWS // NS)],
                    dstg.at[pl.ds(0, DROWS // NS)])
    pltpu.sync_copy(dstg.at[pl.ds(0, DROWS // NS)],
                    den.at[c, pl.ds(d0, DROWS // NS)])


def _sc_edge(h0, h1, a, ei):
    mesh = plsc.VectorSubcoreMesh(
        core_axis_name="c", subcore_axis_name="s",
        num_cores=NC, num_subcores=NS)
    kern = pl.kernel(
        _sc_body,
        out_type=[
            jax.ShapeDtypeStruct((NC, NPAD, HALF), jnp.float32),
            jax.ShapeDtypeStruct((NC, DROWS, HALF), jnp.float32),
        ],
        mesh=mesh,
        compiler_params=pltpu.CompilerParams(needs_layout_passes=False),
        scratch_types=[
            pltpu.VMEM((NPAD,), jnp.float32),
            pltpu.VMEM((NPAD,), jnp.float32),
            pltpu.VMEM((IB,), jnp.int32),
            pltpu.VMEM((IB,), jnp.int32),
            pltpu.VMEM((K,), jnp.int32),
            pltpu.VMEM((2, K), jnp.int32),
            pltpu.VMEM((2, K), jnp.int32),
            pltpu.VMEM((K,), jnp.float32),
            pltpu.VMEM((K,), jnp.int32),
            pltpu.VMEM((K, HALF), jnp.float32),
            pltpu.VMEM((K, HALF), jnp.float32),
            pltpu.VMEM_SHARED((NPAD, HALF), jnp.float32),
            pltpu.VMEM_SHARED((DROWS, HALF), jnp.float32),
            pltpu.SemaphoreType.DMA,
            pltpu.SemaphoreType.DMA,
            pltpu.SemaphoreType.DMA,
        ],
    )
    return kern(h0, h1, a, ei)


def kernel(x, edge_index, W1, att_src1, att_dst1, b1,
           W2, att_src2, att_dst2, b2):
    ei = edge_index.astype(jnp.int32)
    pad = jnp.concatenate(
        [jnp.zeros((1, EPAD - E), jnp.int32),
         jnp.full((1, EPAD - E), NPAD - 1, jnp.int32)], axis=0)
    ei = jnp.concatenate([ei, pad], axis=1)
    h0, h1, a1 = _dense_in(x, W1, att_src1, att_dst1)
    msg1, den1 = _sc_edge(h0, h1, a1, ei)
    h0b, h1b, a2 = _dense_mid(msg1, den1, b1, W2, att_src2, att_dst2)
    msg2, den2 = _sc_edge(h0b, h1b, a2, ei)
    return _dense_out(msg2, den2, b2)
